# baseline (speedup 1.0000x reference)
import functools

import jax
import jax.numpy as jnp
from jax import lax
from jax.experimental import pallas as pl
from jax.experimental.pallas import tpu as pltpu
from jax.experimental.pallas import tpu_sc as plsc

EMBED_D = 32
PACK = 128 // EMBED_D
B_TOTAL = 4096 * 50
NUM_CORES = 2
NUM_SUBCORES = 16
NW = NUM_CORES * NUM_SUBCORES
B_PER_W = B_TOTAL // NW
CHUNK = 320
N_CHUNKS = B_PER_W // CHUNK
LANES = 16

_mesh = plsc.VectorSubcoreMesh(core_axis_name="c", subcore_axis_name="s")


@functools.partial(
    pl.kernel,
    mesh=_mesh,
    out_type=jax.ShapeDtypeStruct((B_TOTAL * EMBED_D,), jnp.float32),
    scratch_types=[
        pltpu.VMEM((B_PER_W,), jnp.int32),
        pltpu.VMEM((B_PER_W,), jnp.int32),
        pltpu.VMEM((CHUNK, 128), jnp.float32),
        pltpu.VMEM((CHUNK, 128), jnp.float32),
        pltpu.VMEM((CHUNK * EMBED_D,), jnp.float32),
        pltpu.VMEM((CHUNK * EMBED_D,), jnp.float32),
        pltpu.SemaphoreType.DMA,
        pltpu.SemaphoreType.DMA,
        pltpu.SemaphoreType.DMA,
        pltpu.SemaphoreType.DMA,
    ],
    compiler_params=pltpu.CompilerParams(
        use_tc_tiling_on_sc=False, needs_layout_passes=False),
)
def _gather_kernel(idx_hbm, tab_hbm, out_hbm, idx_v, grp_v, rows_a, rows_b,
                   outb_a, outb_b, ga_sem, gb_sem, wa_sem, wb_sem):
    wid = lax.axis_index("s") * NUM_CORES + lax.axis_index("c")
    base = wid * B_PER_W
    pltpu.sync_copy(idx_hbm.at[pl.ds(base, B_PER_W)], idx_v)

    def grp_body(i, _):
        sl = pl.ds(i * LANES, LANES)
        grp_v[sl] = lax.shift_right_logical(idx_v[sl], 2)
        return _
    lax.fori_loop(0, B_PER_W // LANES, grp_body, 0)

    rows = (rows_a, rows_b)
    outb = (outb_a, outb_b)
    gsem = (ga_sem, gb_sem)
    wsem = (wa_sem, wb_sem)

    def gather(i):
        p = i % 2
        return pltpu.async_copy(
            tab_hbm.at[grp_v.at[pl.ds(i * CHUNK, CHUNK)]], rows[p], gsem[p])

    lanes = lax.iota(jnp.int32, LANES)
    col_stride = lanes * EMBED_D

    def extract(i):
        p = i % 2

        def m_body(m, _):
            sl = pl.ds(i * CHUNK + m * LANES, LANES)
            sub = (idx_v[sl] & (PACK - 1)) * EMBED_D
            row_ids = m * LANES + lanes
            dst_base = m * (LANES * EMBED_D) + col_stride
            for k in range(EMBED_D):
                v = plsc.load_gather(rows[p], [row_ids, sub + k])
                plsc.store_scatter(outb[p], [dst_base + k], v)
            return _

        lax.fori_loop(0, CHUNK // LANES, m_body, 0)

    def put(i):
        p = i % 2
        off = (base + i * CHUNK) * EMBED_D
        return pltpu.async_copy(
            outb[p], out_hbm.at[pl.ds(off, CHUNK * EMBED_D)], wsem[p])

    writes = [None, None]
    g = [None, None]
    g[0] = gather(0)
    for i in range(N_CHUNKS):
        p = i % 2
        if i + 1 < N_CHUNKS:
            g[1 - p] = gather(i + 1)
        g[p].wait()
        if writes[p] is not None:
            writes[p].wait()
        extract(i)
        writes[p] = put(i)
    for w in writes:
        if w is not None:
            w.wait()


def kernel(x, wordmat):
    idx = x.reshape(-1).astype(jnp.int32)
    tab = wordmat.reshape(wordmat.shape[0] // PACK, 128)
    out = _gather_kernel(idx, tab)
    return out.reshape(x.shape + (EMBED_D,))


# speedup vs baseline: 1.4492x; 1.4492x over previous
import functools

import jax
import jax.numpy as jnp
from jax import lax
from jax.experimental import pallas as pl
from jax.experimental.pallas import tpu as pltpu
from jax.experimental.pallas import tpu_sc as plsc

EMBED_D = 32
B_TOTAL = 4096 * 50
NUM_CORES = 2
NUM_SUBCORES = 16
NW = NUM_CORES * NUM_SUBCORES
B_PER_W = B_TOTAL // NW
CHUNK = 400
N_CHUNKS = B_PER_W // CHUNK
LANES = 16

_mesh = plsc.VectorSubcoreMesh(core_axis_name="c", subcore_axis_name="s")


@functools.partial(
    pl.kernel,
    mesh=_mesh,
    out_type=jax.ShapeDtypeStruct((B_TOTAL, EMBED_D), jnp.float32),
    scratch_types=[
        pltpu.VMEM((B_PER_W,), jnp.int32),
        pltpu.VMEM((CHUNK, EMBED_D), jnp.float32),
        pltpu.VMEM((CHUNK, EMBED_D), jnp.float32),
        pltpu.SemaphoreType.DMA,
        pltpu.SemaphoreType.DMA,
        pltpu.SemaphoreType.DMA,
    ],
    compiler_params=pltpu.CompilerParams(
        use_tc_tiling_on_sc=True, needs_layout_passes=False),
)
def _gather_kernel(idx_hbm, tab_hbm, out_hbm, idx_v, rows_a, rows_b,
                   g_sem, wa_sem, wb_sem):
    wid = lax.axis_index("s") * NUM_CORES + lax.axis_index("c")
    base = wid * B_PER_W
    pltpu.sync_copy(idx_hbm.at[pl.ds(base, B_PER_W)], idx_v)

    rows = (rows_a, rows_b)
    wsem = (wa_sem, wb_sem)

    def fill(i, buf):
        def body(m, _):
            vec = idx_v[pl.ds(i * CHUNK + m * LANES, LANES)]
            for l in range(LANES):
                r = vec[l]
                pltpu.async_copy(tab_hbm.at[pl.ds(r, 1)],
                                 buf.at[pl.ds(m * LANES + l, 1)], g_sem)
            return _
        lax.fori_loop(0, CHUNK // LANES, body, 0)
        pltpu.make_async_copy(tab_hbm.at[pl.ds(0, CHUNK)], buf, g_sem).wait()

    writes = [None, None]
    for i in range(N_CHUNKS):
        p = i % 2
        if writes[p] is not None:
            writes[p].wait()
        fill(i, rows[p])
        writes[p] = pltpu.async_copy(
            rows[p], out_hbm.at[pl.ds(base + i * CHUNK, CHUNK)], wsem[p])
    for w in writes:
        if w is not None:
            w.wait()


def kernel(x, wordmat):
    idx = x.reshape(-1).astype(jnp.int32)
    out = _gather_kernel(idx, wordmat)
    return out.reshape(x.shape + (EMBED_D,))
